# baseline (speedup 1.0000x reference)
import functools

import jax
import jax.numpy as jnp
from jax import lax
from jax.experimental import pallas as pl
from jax.experimental.pallas import tpu as pltpu
from jax.experimental.pallas import tpu_sc as plsc

TABLE_SIZE = 1000000
MEM_DIM = 128
DIM = 2048

_NC = 2
_NS = 16
_NW = _NC * _NS

_BLOCK_ROWS = 1024
_CHUNKS = 4


def _sc_gather(table, idx, n_rows):
    b_per_w = n_rows // _NW
    mesh = plsc.VectorSubcoreMesh(core_axis_name="c", subcore_axis_name="s")

    @functools.partial(
        pl.kernel,
        mesh=mesh,
        out_type=jax.ShapeDtypeStruct((n_rows, MEM_DIM), jnp.float32),
        scratch_types=[
            pltpu.VMEM((b_per_w,), jnp.int32),
            pltpu.VMEM((b_per_w, MEM_DIM), jnp.float32),
            pltpu.SemaphoreType.DMA,
        ],
    )
    def gather_kernel(table_hbm, idx_hbm, out_hbm, idx_v, rows_v, sem):
        wid = lax.axis_index("s") * _NC + lax.axis_index("c")
        base = wid * b_per_w
        pltpu.sync_copy(idx_hbm.at[pl.ds(base, b_per_w)], idx_v)
        pltpu.async_copy(table_hbm.at[idx_v], rows_v, sem).wait()
        pltpu.sync_copy(rows_v, out_hbm.at[pl.ds(base, b_per_w)])

    return gather_kernel(table, idx)


def _mm_body(ids_ref, mem_ref, w_ref, out_ref):
    mask = (ids_ref[0, 0, :] != 0).astype(jnp.float32)
    mem = mem_ref[...] * mask[:, None]
    out_ref[...] = lax.dot_general(
        mem, w_ref[...], (((1,), (1,)), ((), ())),
        preferred_element_type=jnp.float32)


def _mm_body_alias(prev_ref, ids_ref, mem_ref, w_ref, out_ref):
    del prev_ref
    _mm_body(ids_ref, mem_ref, w_ref, out_ref)


def _tc_matmul_chunk(mem, w, ids3, out_prev, n_rows, block_off):
    chunk_blocks = ids3.shape[0]
    grid = (chunk_blocks,)
    data_specs = [
        pl.BlockSpec((1, 1, _BLOCK_ROWS), lambda i: (i, 0, 0)),
        pl.BlockSpec((_BLOCK_ROWS, MEM_DIM), lambda i: (i, 0)),
        pl.BlockSpec((DIM, MEM_DIM), lambda i: (0, 0)),
    ]
    out_spec = pl.BlockSpec((_BLOCK_ROWS, DIM), lambda i: (i + block_off, 0))
    out_shape = jax.ShapeDtypeStruct((n_rows, DIM), jnp.float32)
    if out_prev is None:
        return pl.pallas_call(
            _mm_body, grid=grid, in_specs=data_specs,
            out_specs=out_spec, out_shape=out_shape,
        )(ids3, mem, w)
    return pl.pallas_call(
        _mm_body_alias, grid=grid,
        in_specs=[pl.BlockSpec(memory_space=pl.ANY)] + data_specs,
        out_specs=out_spec, out_shape=out_shape,
        input_output_aliases={0: 0},
    )(out_prev, ids3, mem, w)


def kernel(x, bigram_ids, layer_id, collect_stats, E, W):
    b, s = bigram_ids.shape
    n_rows = b * s
    ids = bigram_ids.reshape(n_rows).astype(jnp.int32)
    rows_per_chunk = n_rows // _CHUNKS
    blocks_per_chunk = rows_per_chunk // _BLOCK_ROWS
    out = None
    for c in range(_CHUNKS):
        ids_c = ids[c * rows_per_chunk:(c + 1) * rows_per_chunk]
        mem_c = _sc_gather(E, ids_c, rows_per_chunk)
        ids3_c = ids_c.reshape(blocks_per_chunk, 1, _BLOCK_ROWS)
        out = _tc_matmul_chunk(mem_c, W, ids3_c, out, n_rows,
                               c * blocks_per_chunk)
    return out.reshape(b, s, DIM)


# speedup vs baseline: 1.1384x; 1.1384x over previous
import functools

import jax
import jax.numpy as jnp
from jax import lax
from jax.experimental import pallas as pl
from jax.experimental.pallas import tpu as pltpu
from jax.experimental.pallas import tpu_sc as plsc

TABLE_SIZE = 1000000
MEM_DIM = 128
DIM = 2048

_NC = 2
_NS = 16
_NW = _NC * _NS

_BLOCK_ROWS = 1024
_CHUNKS = 1
_SC_SUB = 4


def _sc_gather(table, idx, n_rows):
    b_per_w = n_rows // _NW
    mesh = plsc.VectorSubcoreMesh(core_axis_name="c", subcore_axis_name="s")

    sub = b_per_w // _SC_SUB

    @functools.partial(
        pl.kernel,
        mesh=mesh,
        out_type=jax.ShapeDtypeStruct((n_rows, MEM_DIM), jnp.float32),
        scratch_types=[
            pltpu.VMEM((b_per_w,), jnp.int32),
            pltpu.VMEM((b_per_w, MEM_DIM), jnp.float32),
            [pltpu.SemaphoreType.DMA] * _SC_SUB,
            pltpu.SemaphoreType.DMA,
        ],
    )
    def gather_kernel(table_hbm, idx_hbm, out_hbm, idx_v, rows_v, gsems, wsem):
        wid = lax.axis_index("s") * _NC + lax.axis_index("c")
        base = wid * b_per_w
        pltpu.sync_copy(idx_hbm.at[pl.ds(base, b_per_w)], idx_v)
        gathers = [
            pltpu.async_copy(
                table_hbm.at[idx_v.at[pl.ds(s * sub, sub)]],
                rows_v.at[pl.ds(s * sub, sub)], gsems[s])
            for s in range(_SC_SUB)
        ]
        writes = []
        for s in range(_SC_SUB):
            gathers[s].wait()
            writes.append(pltpu.async_copy(
                rows_v.at[pl.ds(s * sub, sub)],
                out_hbm.at[pl.ds(base + s * sub, sub)], wsem))
        for w in writes:
            w.wait()

    return gather_kernel(table, idx)


def _mm_body(ids_ref, mem_ref, w_ref, out_ref):
    mask = (ids_ref[0, 0, :] != 0).astype(jnp.float32)
    mem = mem_ref[...] * mask[:, None]
    out_ref[...] = lax.dot_general(
        mem, w_ref[...], (((1,), (1,)), ((), ())),
        preferred_element_type=jnp.float32)


def _mm_body_alias(prev_ref, ids_ref, mem_ref, w_ref, out_ref):
    del prev_ref
    _mm_body(ids_ref, mem_ref, w_ref, out_ref)


def _tc_matmul_chunk(mem, w, ids3, out_prev, n_rows, block_off):
    chunk_blocks = ids3.shape[0]
    grid = (chunk_blocks,)
    data_specs = [
        pl.BlockSpec((1, 1, _BLOCK_ROWS), lambda i: (i, 0, 0)),
        pl.BlockSpec((_BLOCK_ROWS, MEM_DIM), lambda i: (i, 0)),
        pl.BlockSpec((DIM, MEM_DIM), lambda i: (0, 0)),
    ]
    out_spec = pl.BlockSpec((_BLOCK_ROWS, DIM), lambda i: (i + block_off, 0))
    out_shape = jax.ShapeDtypeStruct((n_rows, DIM), jnp.float32)
    if out_prev is None:
        return pl.pallas_call(
            _mm_body, grid=grid, in_specs=data_specs,
            out_specs=out_spec, out_shape=out_shape,
        )(ids3, mem, w)
    return pl.pallas_call(
        _mm_body_alias, grid=grid,
        in_specs=[pl.BlockSpec(memory_space=pl.ANY)] + data_specs,
        out_specs=out_spec, out_shape=out_shape,
        input_output_aliases={0: 0},
    )(out_prev, ids3, mem, w)


def kernel(x, bigram_ids, layer_id, collect_stats, E, W):
    b, s = bigram_ids.shape
    n_rows = b * s
    ids = bigram_ids.reshape(n_rows).astype(jnp.int32)
    rows_per_chunk = n_rows // _CHUNKS
    blocks_per_chunk = rows_per_chunk // _BLOCK_ROWS
    out = None
    for c in range(_CHUNKS):
        ids_c = ids[c * rows_per_chunk:(c + 1) * rows_per_chunk]
        mem_c = _sc_gather(E, ids_c, rows_per_chunk)
        ids3_c = ids_c.reshape(blocks_per_chunk, 1, _BLOCK_ROWS)
        out = _tc_matmul_chunk(mem_c, W, ids3_c, out, n_rows,
                               c * blocks_per_chunk)
    return out.reshape(b, s, DIM)


# speedup vs baseline: 1.1414x; 1.0026x over previous
import functools

import jax
import jax.numpy as jnp
from jax import lax
from jax.experimental import pallas as pl
from jax.experimental.pallas import tpu as pltpu
from jax.experimental.pallas import tpu_sc as plsc

TABLE_SIZE = 1000000
MEM_DIM = 128
DIM = 2048

_NC = 2
_NS = 16
_NW = _NC * _NS

_BLOCK_ROWS = 1024
_CHUNKS = 1
_SC_SUB = 2


def _sc_gather(table, idx, n_rows):
    b_per_w = n_rows // _NW
    mesh = plsc.VectorSubcoreMesh(core_axis_name="c", subcore_axis_name="s")

    sub = b_per_w // _SC_SUB

    @functools.partial(
        pl.kernel,
        mesh=mesh,
        out_type=jax.ShapeDtypeStruct((n_rows, MEM_DIM), jnp.float32),
        scratch_types=[
            pltpu.VMEM((b_per_w,), jnp.int32),
            pltpu.VMEM((b_per_w, MEM_DIM), jnp.float32),
            [pltpu.SemaphoreType.DMA] * _SC_SUB,
            pltpu.SemaphoreType.DMA,
        ],
    )
    def gather_kernel(table_hbm, idx_hbm, out_hbm, idx_v, rows_v, gsems, wsem):
        wid = lax.axis_index("s") * _NC + lax.axis_index("c")
        base = wid * b_per_w
        pltpu.sync_copy(idx_hbm.at[pl.ds(base, b_per_w)], idx_v)
        gathers = [
            pltpu.async_copy(
                table_hbm.at[idx_v.at[pl.ds(s * sub, sub)]],
                rows_v.at[pl.ds(s * sub, sub)], gsems[s])
            for s in range(_SC_SUB)
        ]
        writes = []
        for s in range(_SC_SUB):
            gathers[s].wait()
            writes.append(pltpu.async_copy(
                rows_v.at[pl.ds(s * sub, sub)],
                out_hbm.at[pl.ds(base + s * sub, sub)], wsem))
        for w in writes:
            w.wait()

    return gather_kernel(table, idx)


def _mm_body(ids_ref, mem_ref, w_ref, out_ref):
    mask = (ids_ref[0, 0, :] != 0).astype(jnp.float32)
    mem = mem_ref[...] * mask[:, None]
    out_ref[...] = lax.dot_general(
        mem, w_ref[...], (((1,), (1,)), ((), ())),
        preferred_element_type=jnp.float32)


def _mm_body_alias(prev_ref, ids_ref, mem_ref, w_ref, out_ref):
    del prev_ref
    _mm_body(ids_ref, mem_ref, w_ref, out_ref)


def _tc_matmul_chunk(mem, w, ids3, out_prev, n_rows, block_off):
    chunk_blocks = ids3.shape[0]
    grid = (chunk_blocks,)
    data_specs = [
        pl.BlockSpec((1, 1, _BLOCK_ROWS), lambda i: (i, 0, 0)),
        pl.BlockSpec((_BLOCK_ROWS, MEM_DIM), lambda i: (i, 0)),
        pl.BlockSpec((DIM, MEM_DIM), lambda i: (0, 0)),
    ]
    out_spec = pl.BlockSpec((_BLOCK_ROWS, DIM), lambda i: (i + block_off, 0))
    out_shape = jax.ShapeDtypeStruct((n_rows, DIM), jnp.float32)
    if out_prev is None:
        return pl.pallas_call(
            _mm_body, grid=grid, in_specs=data_specs,
            out_specs=out_spec, out_shape=out_shape,
        )(ids3, mem, w)
    return pl.pallas_call(
        _mm_body_alias, grid=grid,
        in_specs=[pl.BlockSpec(memory_space=pl.ANY)] + data_specs,
        out_specs=out_spec, out_shape=out_shape,
        input_output_aliases={0: 0},
    )(out_prev, ids3, mem, w)


def kernel(x, bigram_ids, layer_id, collect_stats, E, W):
    b, s = bigram_ids.shape
    n_rows = b * s
    ids = bigram_ids.reshape(n_rows).astype(jnp.int32)
    rows_per_chunk = n_rows // _CHUNKS
    blocks_per_chunk = rows_per_chunk // _BLOCK_ROWS
    out = None
    for c in range(_CHUNKS):
        ids_c = ids[c * rows_per_chunk:(c + 1) * rows_per_chunk]
        mem_c = _sc_gather(E, ids_c, rows_per_chunk)
        ids3_c = ids_c.reshape(blocks_per_chunk, 1, _BLOCK_ROWS)
        out = _tc_matmul_chunk(mem_c, W, ids3_c, out, n_rows,
                               c * blocks_per_chunk)
    return out.reshape(b, s, DIM)
